# Initial kernel scaffold; baseline (speedup 1.0000x reference)
import jax
import jax.numpy as jnp
from jax.experimental import pallas as pl


def kernel(x, weight, bias):
    raise NotImplementedError("write your pallas kernel here")


# speedup vs baseline: 2.3917x; 2.3917x over previous
import jax
import jax.numpy as jnp
from jax.experimental import pallas as pl

_TOKENS = 8192
_DIM = 7168
_NE = 384
_K = 6
_SCALE = 2.5
_BT = 256


def _gate_body(x_ref, wt_ref, bias_ref, w_out_ref, i_out_ref):
    x = x_ref[...]
    wt = wt_ref[...]
    scores = jax.lax.dot_general(
        x, wt, (((1,), (0,)), ((), ())),
        preferred_element_type=jnp.float32)
    s = jnp.sqrt(jax.nn.softplus(scores))
    biased = s + bias_ref[...]
    iota = jax.lax.broadcasted_iota(jnp.int32, (_BT, _NE), 1)
    cur = biased
    vals = []
    for j in range(_K):
        m = jnp.max(cur, axis=1, keepdims=True)
        idx = jnp.min(jnp.where(cur == m, iota, _NE), axis=1)
        sel = iota == idx[:, None]
        vals.append(jnp.sum(jnp.where(sel, s, 0.0), axis=1))
        i_out_ref[j, :] = idx
        cur = jnp.where(sel, -jnp.inf, cur)
    inv = _SCALE / (vals[0] + vals[1] + vals[2] + vals[3] + vals[4] + vals[5])
    for j in range(_K):
        w_out_ref[j, :] = vals[j] * inv
    zf = jnp.zeros((_BT,), jnp.float32)
    zi = jnp.zeros((_BT,), jnp.int32)
    for j in range(_K, 8):
        w_out_ref[j, :] = zf
        i_out_ref[j, :] = zi


def kernel(x, weight, bias):
    wt = weight.T
    bias2 = bias.reshape(1, _NE)
    w_out, i_out = pl.pallas_call(
        _gate_body,
        grid=(_TOKENS // _BT,),
        in_specs=[
            pl.BlockSpec((_BT, _DIM), lambda i: (i, 0)),
            pl.BlockSpec((_DIM, _NE), lambda i: (0, 0)),
            pl.BlockSpec((1, _NE), lambda i: (0, 0)),
        ],
        out_specs=[
            pl.BlockSpec((8, _BT), lambda i: (0, i)),
            pl.BlockSpec((8, _BT), lambda i: (0, i)),
        ],
        out_shape=[
            jax.ShapeDtypeStruct((8, _TOKENS), jnp.float32),
            jax.ShapeDtypeStruct((8, _TOKENS), jnp.int32),
        ],
    )(x, wt, bias2)
    return w_out[:_K].T, i_out[:_K].T


# speedup vs baseline: 3.0791x; 1.2874x over previous
import jax
import jax.numpy as jnp
from jax.experimental import pallas as pl

_TOKENS = 8192
_DIM = 7168
_NE = 384
_K = 6
_SCALE = 2.5
_BT = 256


_NSUB = 2
_BS = _BT // _NSUB


def _topk_rows(s, biased):
    iota = jax.lax.broadcasted_iota(jnp.int32, (_BS, _NE), 1)
    cur = biased
    vals, idxs = [], []
    for j in range(_K):
        m = jnp.max(cur, axis=1, keepdims=True)
        idx = jnp.min(jnp.where(cur == m, iota, _NE), axis=1)
        sel = iota == idx[:, None]
        vals.append(jnp.sum(jnp.where(sel, s, 0.0), axis=1))
        idxs.append(idx)
        if j + 1 < _K:
            cur = jnp.where(sel, -jnp.inf, cur)
    return idxs, vals


def _gate_body(x_ref, wt_ref, bias_ref, w_out_ref, i_out_ref):
    wt = wt_ref[...]
    bias = bias_ref[...]
    scs = []
    for h in range(_NSUB):
        x = x_ref[h * _BS:(h + 1) * _BS, :]
        scs.append(jax.lax.dot_general(
            x, wt, (((1,), (0,)), ((), ())),
            preferred_element_type=jnp.float32))
    for h in range(_NSUB):
        s = jnp.sqrt(jax.nn.softplus(scs[h]))
        idxs, vals = _topk_rows(s, s + bias)
        inv = _SCALE / (vals[0] + vals[1] + vals[2]
                        + vals[3] + vals[4] + vals[5])
        col = pl.ds(h * _BS, _BS)
        for j in range(_K):
            i_out_ref[j, col] = idxs[j]
            w_out_ref[j, col] = vals[j] * inv
        zf = jnp.zeros((_BS,), jnp.float32)
        zi = jnp.zeros((_BS,), jnp.int32)
        for j in range(_K, 8):
            w_out_ref[j, col] = zf
            i_out_ref[j, col] = zi


def kernel(x, weight, bias):
    wt = weight.T
    bias2 = bias.reshape(1, _NE)
    w_out, i_out = pl.pallas_call(
        _gate_body,
        grid=(_TOKENS // _BT,),
        in_specs=[
            pl.BlockSpec((_BT, _DIM), lambda i: (i, 0)),
            pl.BlockSpec((_DIM, _NE), lambda i: (0, 0)),
            pl.BlockSpec((1, _NE), lambda i: (0, 0)),
        ],
        out_specs=[
            pl.BlockSpec((8, _BT), lambda i: (0, i)),
            pl.BlockSpec((8, _BT), lambda i: (0, i)),
        ],
        out_shape=[
            jax.ShapeDtypeStruct((8, _TOKENS), jnp.float32),
            jax.ShapeDtypeStruct((8, _TOKENS), jnp.int32),
        ],
    )(x, wt, bias2)
    return w_out[:_K].T, i_out[:_K].T


# speedup vs baseline: 3.7548x; 1.2194x over previous
import jax
import jax.numpy as jnp
from jax.experimental import pallas as pl

_TOKENS = 8192
_DIM = 7168
_NE = 384
_K = 6
_SCALE = 2.5
_BT = 512


_NSUB = 4
_BS = _BT // _NSUB


def _topk_rows(s, biased):
    iota = jax.lax.broadcasted_iota(jnp.int32, (_BS, _NE), 1)
    cur = biased
    vals, idxs = [], []
    for j in range(_K):
        m = jnp.max(cur, axis=1, keepdims=True)
        idx = jnp.min(jnp.where(cur == m, iota, _NE), axis=1)
        sel = iota == idx[:, None]
        vals.append(jnp.sum(jnp.where(sel, s, 0.0), axis=1))
        idxs.append(idx)
        if j + 1 < _K:
            cur = jnp.where(sel, -jnp.inf, cur)
    return idxs, vals


def _gate_body(x_ref, wt_ref, bias_ref, w_out_ref, i_out_ref):
    wt = wt_ref[...]
    bias = bias_ref[...]
    scs = []
    for h in range(_NSUB):
        x = x_ref[h * _BS:(h + 1) * _BS, :]
        scs.append(jax.lax.dot_general(
            x, wt, (((1,), (0,)), ((), ())),
            preferred_element_type=jnp.float32))
    for h in range(_NSUB):
        s = jnp.sqrt(jax.nn.softplus(scs[h]))
        idxs, vals = _topk_rows(s, s + bias)
        inv = _SCALE / (vals[0] + vals[1] + vals[2]
                        + vals[3] + vals[4] + vals[5])
        col = pl.ds(h * _BS, _BS)
        for j in range(_K):
            i_out_ref[j, col] = idxs[j]
            w_out_ref[j, col] = vals[j] * inv
        zf = jnp.zeros((_BS,), jnp.float32)
        zi = jnp.zeros((_BS,), jnp.int32)
        for j in range(_K, 8):
            w_out_ref[j, col] = zf
            i_out_ref[j, col] = zi


def kernel(x, weight, bias):
    wt = weight.T
    bias2 = bias.reshape(1, _NE)
    w_out, i_out = pl.pallas_call(
        _gate_body,
        grid=(_TOKENS // _BT,),
        in_specs=[
            pl.BlockSpec((_BT, _DIM), lambda i: (i, 0)),
            pl.BlockSpec((_DIM, _NE), lambda i: (0, 0)),
            pl.BlockSpec((1, _NE), lambda i: (0, 0)),
        ],
        out_specs=[
            pl.BlockSpec((8, _BT), lambda i: (0, i)),
            pl.BlockSpec((8, _BT), lambda i: (0, i)),
        ],
        out_shape=[
            jax.ShapeDtypeStruct((8, _TOKENS), jnp.float32),
            jax.ShapeDtypeStruct((8, _TOKENS), jnp.int32),
        ],
    )(x, wt, bias2)
    return w_out[:_K].T, i_out[:_K].T
